# baseline (speedup 1.0000x reference)
import jax
import jax.numpy as jnp
from jax import lax
from jax.experimental import pallas as pl
from jax.experimental.pallas import tpu as pltpu
from jax.experimental.pallas import tpu_sc as plsc

N = 200000
C = 32
K = 27
EPS = 1e-5

NC = 2
NS = 16
NW = NC * NS

R = 64
CH = N // R
CPW_MAX = -(-CH // NW)
REM = CH - (CPW_MAX - 1) * NW

MB = 800
EB = 8000


def _matmul_split_body(x_ref, w_ref, *o_refs):
    y = jnp.dot(x_ref[...], w_ref[...], preferred_element_type=jnp.float32)
    for k in range(K):
        o_refs[k][...] = y[:, k * C:(k + 1) * C]


def _tc_matmul_split(x, w_mat):
    return pl.pallas_call(
        _matmul_split_body,
        grid=(N // MB,),
        in_specs=[
            pl.BlockSpec((MB, C), lambda i: (i, 0)),
            pl.BlockSpec((C, K * C), lambda i: (0, 0)),
        ],
        out_specs=[pl.BlockSpec((MB, C), lambda i: (i, 0))] * K,
        out_shape=[jax.ShapeDtypeStruct((N, C), jnp.float32)] * K,
    )(x, w_mat)


def _gather_sum_body(*refs):
    tables = refs[:K]
    idx_hbm = refs[K]
    out_hbm = refs[K + 1]
    stats_hbm = refs[K + 2]
    idx_v, buf, acc, stats_v, sem = refs[K + 3:]

    wid = lax.axis_index("s") * NC + lax.axis_index("c")
    n_chunks = CPW_MAX - 1 + jnp.where(wid < REM, 1, 0)

    def chunk(j, carry):
        s0, s1, q0, q1 = carry
        c = j * NW + wid
        pltpu.sync_copy(idx_hbm.at[c], idx_v)
        copies = [
            pltpu.async_copy(tables[k].at[idx_v.at[k]],
                             buf.at[pl.ds(k * R, R)], sem)
            for k in range(K)
        ]
        for cp in copies:
            cp.wait()

        def site(n, carry2):
            s0, s1, q0, q1 = carry2
            a0 = buf[n, pl.ds(0, 16)]
            a1 = buf[n, pl.ds(16, 16)]
            for k in range(1, K):
                a0 = a0 + buf[k * R + n, pl.ds(0, 16)]
                a1 = a1 + buf[k * R + n, pl.ds(16, 16)]
            acc[n, pl.ds(0, 16)] = a0
            acc[n, pl.ds(16, 16)] = a1
            return (s0 + a0, s1 + a1, q0 + a0 * a0, q1 + a1 * a1)

        carry = lax.fori_loop(0, R, site, (s0, s1, q0, q1), unroll=False)
        pltpu.sync_copy(acc, out_hbm.at[pl.ds(c * R, R)])
        return carry

    z = jnp.zeros((16,), jnp.float32)
    s0, s1, q0, q1 = lax.fori_loop(0, n_chunks, chunk, (z, z, z, z),
                                   unroll=False)
    stats_v[pl.ds(0, 16)] = s0
    stats_v[pl.ds(16, 16)] = s1
    stats_v[pl.ds(32, 16)] = q0
    stats_v[pl.ds(48, 16)] = q1
    pltpu.sync_copy(stats_v, stats_hbm.at[wid])


def _sc_gather_sum(tables, idx3):
    mesh = plsc.VectorSubcoreMesh(core_axis_name="c", subcore_axis_name="s",
                                  num_cores=NC, num_subcores=NS)
    out, stats = pl.kernel(
        _gather_sum_body,
        out_type=[
            jax.ShapeDtypeStruct((N, C), jnp.float32),
            jax.ShapeDtypeStruct((NW, 2 * C), jnp.float32),
        ],
        mesh=mesh,
        scratch_types=[
            pltpu.VMEM((K, R), jnp.int32),
            pltpu.VMEM((K * R, C), jnp.float32),
            pltpu.VMEM((R, C), jnp.float32),
            pltpu.VMEM((2 * C,), jnp.float32),
            pltpu.SemaphoreType.DMA,
        ],
        compiler_params=pltpu.CompilerParams(use_tc_tiling_on_sc=False),
    )(*tables, idx3)
    return out, stats


def _bn_scale_shift(stats_ref, g_ref, b_ref):
    s = jnp.sum(stats_ref[...], axis=0)
    mean = s[:C] * (1.0 / N)
    var = s[C:] * (1.0 / N) - mean * mean
    scale = g_ref[...] * lax.rsqrt(var + EPS)
    shift = b_ref[...] - mean * scale
    return scale, shift


def _bn_relu_matmul_body(h_ref, stats_ref, g_ref, b_ref, w_ref, *o_refs):
    scale, shift = _bn_scale_shift(stats_ref, g_ref, b_ref)
    z = jnp.maximum(h_ref[...] * scale[None, :] + shift[None, :], 0.0)
    y = jnp.dot(z, w_ref[...], preferred_element_type=jnp.float32)
    for k in range(K):
        o_refs[k][...] = y[:, k * C:(k + 1) * C]


def _tc_bn_relu_matmul(h, stats, gamma, beta, w_mat):
    return pl.pallas_call(
        _bn_relu_matmul_body,
        grid=(N // MB,),
        in_specs=[
            pl.BlockSpec((MB, C), lambda i: (i, 0)),
            pl.BlockSpec((NW, 2 * C), lambda i: (0, 0)),
            pl.BlockSpec((C,), lambda i: (0,)),
            pl.BlockSpec((C,), lambda i: (0,)),
            pl.BlockSpec((C, K * C), lambda i: (0, 0)),
        ],
        out_specs=[pl.BlockSpec((MB, C), lambda i: (i, 0))] * K,
        out_shape=[jax.ShapeDtypeStruct((N, C), jnp.float32)] * K,
    )(h, stats, gamma, beta, w_mat)


def _bn_res_relu_body(h_ref, stats_ref, g_ref, b_ref, x_ref, o_ref):
    scale, shift = _bn_scale_shift(stats_ref, g_ref, b_ref)
    o_ref[...] = jnp.maximum(
        h_ref[...] * scale[None, :] + shift[None, :] + x_ref[...], 0.0)


def _tc_bn_res_relu(h, stats, gamma, beta, x):
    return pl.pallas_call(
        _bn_res_relu_body,
        grid=(N // EB,),
        in_specs=[
            pl.BlockSpec((EB, C), lambda i: (i, 0)),
            pl.BlockSpec((NW, 2 * C), lambda i: (0, 0)),
            pl.BlockSpec((C,), lambda i: (0,)),
            pl.BlockSpec((C,), lambda i: (0,)),
            pl.BlockSpec((EB, C), lambda i: (i, 0)),
        ],
        out_specs=pl.BlockSpec((EB, C), lambda i: (i, 0)),
        out_shape=jax.ShapeDtypeStruct((N, C), jnp.float32),
    )(h, stats, gamma, beta, x)


@jax.jit
def kernel(x, nbr_idx, W1, gamma1, beta1, W2, gamma2, beta2):
    w1m = W1.transpose(1, 0, 2).reshape(C, K * C)
    w2m = W2.transpose(1, 0, 2).reshape(C, K * C)

    idx3 = nbr_idx.reshape(CH, R, K).transpose(0, 2, 1)

    y1 = _tc_matmul_split(x, w1m)
    h1, st1 = _sc_gather_sum(y1, idx3)
    y2 = _tc_bn_relu_matmul(h1, st1, gamma1, beta1, w2m)
    h2, st2 = _sc_gather_sum(y2, idx3)
    return _tc_bn_res_relu(h2, st2, gamma2, beta2, x)
